# baseline (speedup 1.0000x reference)
import functools

import jax
import jax.numpy as jnp
from jax import lax
from jax.experimental import pallas as pl
from jax.experimental.pallas import tpu as pltpu
from jax.experimental.pallas import tpu_sc as plsc

E = 8
D = 64
T = 8192
C = 1024
CH = 1024
NSTEP = T // CH
NSLOT = 9 * 1024
TRASH = E * C

NW2 = 16
TPW2 = T // NW2
NW4 = 32
TPW4 = T // NW4


def _router_body(x_ref, wr_ref, br_ref, xg_ref, gate_ref, idx_ref):
    x = x_ref[...]
    logits = jnp.dot(x, wr_ref[...],
                     preferred_element_type=jnp.float32) + br_ref[...]
    m = jnp.max(logits, axis=-1, keepdims=True)
    denom = jnp.sum(jnp.exp(logits - m), axis=-1, keepdims=True)
    gate = 1.0 / denom
    lane = jax.lax.broadcasted_iota(jnp.int32, (CH, E), 1)
    idx = jnp.min(jnp.where(logits >= m, lane, E), axis=-1,
                  keepdims=True)
    xg_ref[...] = x * gate
    gate_ref[...] = gate
    idx_ref[...] = idx


def _router(x, W_route, b_route):
    return pl.pallas_call(
        _router_body,
        grid=(NSTEP,),
        in_specs=[
            pl.BlockSpec((CH, D), lambda i: (i, 0)),
            pl.BlockSpec((D, E), lambda i: (0, 0)),
            pl.BlockSpec((1, E), lambda i: (0, 0)),
        ],
        out_specs=[
            pl.BlockSpec((CH, D), lambda i: (i, 0)),
            pl.BlockSpec((CH, 1), lambda i: (i, 0)),
            pl.BlockSpec((CH, 1), lambda i: (i, 0)),
        ],
        out_shape=[
            jax.ShapeDtypeStruct((T, D), jnp.float32),
            jax.ShapeDtypeStruct((T, 1), jnp.float32),
            jax.ShapeDtypeStruct((T, 1), jnp.int32),
        ],
    )(x, W_route, b_route.reshape(1, E))


def _dispatch_kernel():
    mesh = plsc.VectorSubcoreMesh(core_axis_name="c", subcore_axis_name="s")

    @functools.partial(
        pl.kernel,
        mesh=mesh,
        out_type=[
            jax.ShapeDtypeStruct((NSLOT, D), jnp.float32),
            jax.ShapeDtypeStruct((NW2, 4, 128), jnp.int32),
            jax.ShapeDtypeStruct((NSLOT,), jnp.float32),
        ],
        scratch_types=[
            pltpu.VMEM((TPW2,), jnp.int32),
            pltpu.VMEM((TPW2,), jnp.float32),
            pltpu.VMEM((TPW2, D), jnp.float32),
            pltpu.VMEM((TPW2,), jnp.int32),
            pltpu.VMEM((TPW2,), jnp.float32),
            pltpu.VMEM((4, 128), jnp.int32),
            pltpu.VMEM((4, 128), jnp.float32),
            pltpu.VMEM((16,), jnp.int32),
            pltpu.VMEM((NW2, 16), jnp.int32),
            pltpu.VMEM_SHARED((NW2, 16), jnp.int32),
        ],
        compiler_params=pltpu.CompilerParams(needs_layout_passes=False, use_tc_tiling_on_sc=False),
    )
    def k2(idx_hbm, gate_hbm, xg_hbm, xslots_hbm, dst_hbm, gslot_hbm,
           idx_v, gate_v, xg_v, dst_lin, gk_lin, dst_v, gk_v,
           hist_v, allhist_v, shared):
        c = lax.axis_index("c")
        s = lax.axis_index("s")
        lanes = lax.iota(jnp.int32, 16)

        @pl.when(c == 0)
        def _():
            pltpu.sync_copy(idx_hbm.at[s], idx_v)

            def hist_step(k, cnts):
                v = idx_v[pl.ds(pl.multiple_of(k * 16, 16), 16)]
                return tuple(
                    cnts[e] + jnp.sum(jnp.where(v == e, 1, 0))
                    for e in range(E))

            zeros = jnp.zeros((16,), jnp.int32)
            cnts = lax.fori_loop(0, TPW2 // 16, hist_step, (zeros,) * E)
            packed = jnp.zeros((16,), jnp.int32)
            for e in range(E):
                packed = jnp.where(lanes == e, cnts[e], packed)
            hist_v[...] = packed
            pltpu.sync_copy(hist_v, shared.at[s])
            plsc.subcore_barrier()
            pltpu.sync_copy(shared, allhist_v)

            base = jnp.zeros((16,), jnp.int32)
            for w in range(NW2):
                base = base + jnp.where(w < s, allhist_v[w, :], 0)
            bases = tuple(
                jnp.zeros((16,), jnp.int32)
                + jnp.sum(jnp.where(lanes == e, base, 0)) for e in range(E))

            pltpu.sync_copy(gate_hbm.at[s], gate_v)
            pltpu.sync_copy(xg_hbm.at[pl.ds(s * TPW2, TPW2)], xg_v)

            def assign_step(k, carry):
                cnt = carry
                off = pl.multiple_of(k * 16, 16)
                v = idx_v[pl.ds(off, 16)]
                gv = gate_v[pl.ds(off, 16)]
                pos = jnp.zeros((16,), jnp.int32)
                new = []
                for e in range(E):
                    msk = v == e
                    cs = plsc.cumsum(jnp.where(msk, 1, 0))
                    pos = jnp.where(msk, cnt[e] + cs, pos)
                    new.append(cnt[e] + jnp.sum(jnp.where(msk, 1, 0)))
                kept = pos < C
                dstv = jnp.where(kept, v * C + pos, TRASH)
                dst_lin[pl.ds(off, 16)] = dstv
                gk_lin[pl.ds(off, 16)] = jnp.where(kept, gv, 0.0)
                return tuple(new)

            lax.fori_loop(0, TPW2 // 16, assign_step, bases)

            for r in range(4):
                for cc in range(8):
                    dst_v[r, pl.ds(cc * 16, 16)] = (
                        dst_lin[pl.ds((r * 8 + cc) * 16, 16)])
                    gk_v[r, pl.ds(cc * 16, 16)] = (
                        gk_lin[pl.ds((r * 8 + cc) * 16, 16)])

            pltpu.sync_copy(dst_v, dst_hbm.at[s])
            for j in range(4):
                pltpu.sync_copy(xg_v.at[pl.ds(j * 128, 128)],
                                xslots_hbm.at[dst_v.at[j]])
                pltpu.sync_copy(gk_v.at[j], gslot_hbm.at[dst_v.at[j]])

    return k2


def _ffn_body(x_ref, g_ref, w1_ref, b1_ref, w2_ref, b2_ref, o_ref):
    x = x_ref[...]
    g = g_ref[...]
    h = jnp.maximum(
        jnp.dot(x, w1_ref[0], preferred_element_type=jnp.float32)
        + b1_ref[0], 0.0)
    o_ref[...] = (jnp.dot(h, w2_ref[0],
                          preferred_element_type=jnp.float32)
                  + b2_ref[0]) * g


def _ffn(xslots, gslot, W1, b1, W2, b2):
    def wmap(i):
        return (jnp.minimum(i, E - 1), 0, 0)

    def bmap(i):
        return (jnp.minimum(i, E - 1), 0, 0)

    return pl.pallas_call(
        _ffn_body,
        grid=(NSLOT // C,),
        in_specs=[
            pl.BlockSpec((C, D), lambda i: (i, 0)),
            pl.BlockSpec((C, 1), lambda i: (i, 0)),
            pl.BlockSpec((1, D, D), wmap),
            pl.BlockSpec((1, 1, D), bmap),
            pl.BlockSpec((1, D, D), wmap),
            pl.BlockSpec((1, 1, D), bmap),
        ],
        out_specs=pl.BlockSpec((C, D), lambda i: (i, 0)),
        out_shape=jax.ShapeDtypeStruct((NSLOT, D), jnp.float32),
    )(xslots, gslot.reshape(NSLOT, 1), W1, b1.reshape(E, 1, D),
      W2, b2.reshape(E, 1, D))


def _combine_kernel():
    mesh = plsc.VectorSubcoreMesh(core_axis_name="c", subcore_axis_name="s")

    @functools.partial(
        pl.kernel,
        mesh=mesh,
        out_type=jax.ShapeDtypeStruct((T, D), jnp.float32),
        scratch_types=[
            pltpu.VMEM((2, 128), jnp.int32),
            pltpu.VMEM((TPW4, D), jnp.float32),
            pltpu.SemaphoreType.DMA,
            pltpu.SemaphoreType.DMA,
        ],
        compiler_params=pltpu.CompilerParams(needs_layout_passes=False, use_tc_tiling_on_sc=False),
    )
    def k4(dst_hbm, eo_hbm, out_hbm, dst_v, rows_v, sem0, sem1):
        c = lax.axis_index("c")
        s = lax.axis_index("s")
        wid = s * 2 + c
        pltpu.sync_copy(dst_hbm.at[wid], dst_v)
        cp0 = pltpu.async_copy(eo_hbm.at[dst_v.at[0]],
                               rows_v.at[pl.ds(0, 128)], sem0)
        cp1 = pltpu.async_copy(eo_hbm.at[dst_v.at[1]],
                               rows_v.at[pl.ds(128, 128)], sem1)
        cp0.wait()
        cp1.wait()
        pltpu.sync_copy(rows_v, out_hbm.at[pl.ds(wid * TPW4, TPW4)])

    return k4


def kernel(inputs, W_route, b_route, W1, b1, W2, b2):
    x = inputs.reshape(T, D)
    xg, gate, idx = _router(x, W_route, b_route)
    xslots, dst, gslot = _dispatch_kernel()(
        idx.reshape(NW2, TPW2), gate.reshape(NW2, TPW2), xg)
    eo = _ffn(xslots, gslot, W1, b1, W2, b2)
    out = _combine_kernel()(dst.reshape(NW4, 2, 128), eo)
    return out.reshape(inputs.shape)


# speedup vs baseline: 4.9198x; 4.9198x over previous
import jax
import jax.numpy as jnp
from jax.experimental import pallas as pl
from jax.experimental.pallas import tpu as pltpu

E = 8
D = 64
T = 8192
C = 1024
CH = 1024
NSTEP = T // CH


def _body(x_ref, wr_ref, br_ref, w1_ref, b1_ref, w2_ref, b2_ref, l_ref,
          o_ref, cnt_ref):
    i = pl.program_id(0)

    @pl.when(i == 0)
    def _init():
        cnt_ref[...] = jnp.zeros_like(cnt_ref)

    x = x_ref[...]
    logits = jnp.dot(x, wr_ref[...],
                     preferred_element_type=jnp.float32) + br_ref[...]
    m = jnp.max(logits, axis=-1, keepdims=True)
    denom = jnp.sum(jnp.exp(logits - m), axis=-1, keepdims=True)
    gate = 1.0 / denom
    lane = jax.lax.broadcasted_iota(jnp.int32, (CH, E), 1)
    idx = jnp.min(jnp.where(logits >= m, lane, E), axis=-1,
                  keepdims=True)
    onehot = (lane == idx).astype(jnp.float32)

    csum = jnp.dot(l_ref[...], onehot, preferred_element_type=jnp.float32)
    pos = jnp.sum((csum + cnt_ref[...]) * onehot, axis=-1, keepdims=True)
    cnt_ref[...] = cnt_ref[...] + jnp.sum(onehot, axis=0, keepdims=True)
    kept = (pos < float(C)).astype(jnp.float32)
    g = gate * kept

    h = jnp.maximum(
        jnp.dot(x * g, w1_ref[...], preferred_element_type=jnp.float32)
        + b1_ref[...], 0.0)
    lane_e = jax.lax.broadcasted_iota(jnp.int32, (CH, E * D), 1) // D
    hm = jnp.where(lane_e == idx, h, 0.0) * g
    o = jnp.dot(hm, w2_ref[...], preferred_element_type=jnp.float32)
    b2sel = jnp.dot(onehot, b2_ref[...], preferred_element_type=jnp.float32)
    o_ref[...] = o + b2sel * g


def kernel(inputs, W_route, b_route, W1, b1, W2, b2):
    x = inputs.reshape(T, D)
    w1c = W1.transpose(1, 0, 2).reshape(D, E * D)
    b1c = b1.reshape(1, E * D)
    w2c = W2.reshape(E * D, D)
    tril = jnp.tril(jnp.ones((CH, CH), jnp.float32))

    out = pl.pallas_call(
        _body,
        grid=(NSTEP,),
        in_specs=[
            pl.BlockSpec((CH, D), lambda i: (i, 0)),
            pl.BlockSpec((D, E), lambda i: (0, 0)),
            pl.BlockSpec((1, E), lambda i: (0, 0)),
            pl.BlockSpec((D, E * D), lambda i: (0, 0)),
            pl.BlockSpec((1, E * D), lambda i: (0, 0)),
            pl.BlockSpec((E * D, D), lambda i: (0, 0)),
            pl.BlockSpec((E, D), lambda i: (0, 0)),
            pl.BlockSpec((CH, CH), lambda i: (0, 0)),
        ],
        out_specs=pl.BlockSpec((CH, D), lambda i: (i, 0)),
        out_shape=jax.ShapeDtypeStruct((T, D), jnp.float32),
        scratch_shapes=[pltpu.VMEM((1, E), jnp.float32)],
        compiler_params=pltpu.CompilerParams(
            dimension_semantics=("arbitrary",)),
    )(x, W_route, b_route.reshape(1, E), w1c, b1c, w2c, b2, tril)
    return out.reshape(inputs.shape)


# speedup vs baseline: 5.1727x; 1.0514x over previous
import jax
import jax.numpy as jnp
from jax.experimental import pallas as pl
from jax.experimental.pallas import tpu as pltpu

E = 8
D = 64
T = 8192
C = 1024
CH = 512
NSTEP = T // CH


def _body(x_ref, wr_ref, br_ref, w1_ref, b1_ref, w2_ref, b2_ref, l_ref,
          o_ref, cnt_ref):
    i = pl.program_id(0)

    @pl.when(i == 0)
    def _init():
        cnt_ref[...] = jnp.zeros_like(cnt_ref)

    x = x_ref[...]
    logits = jnp.dot(x, wr_ref[...],
                     preferred_element_type=jnp.float32) + br_ref[...]
    m = jnp.max(logits, axis=-1, keepdims=True)
    denom = jnp.sum(jnp.exp(logits - m), axis=-1, keepdims=True)
    gate = 1.0 / denom
    lane = jax.lax.broadcasted_iota(jnp.int32, (CH, E), 1)
    idx = jnp.min(jnp.where(logits >= m, lane, E), axis=-1,
                  keepdims=True)
    onehot = (lane == idx).astype(jnp.float32)

    csum = jnp.dot(l_ref[...], onehot, preferred_element_type=jnp.float32)
    pos = jnp.sum((csum + cnt_ref[...]) * onehot, axis=-1, keepdims=True)
    cnt_ref[...] = cnt_ref[...] + jnp.sum(onehot, axis=0, keepdims=True)
    kept = (pos < float(C)).astype(jnp.float32)
    g = gate * kept

    h = jnp.maximum(
        jnp.dot(x * g, w1_ref[...], preferred_element_type=jnp.float32)
        + b1_ref[...], 0.0)
    lane_e = jax.lax.broadcasted_iota(jnp.int32, (CH, E * D), 1) // D
    hm = jnp.where(lane_e == idx, h, 0.0) * g
    o = jnp.dot(hm, w2_ref[...], preferred_element_type=jnp.float32)
    b2sel = jnp.dot(onehot, b2_ref[...], preferred_element_type=jnp.float32)
    o_ref[...] = o + b2sel * g


def kernel(inputs, W_route, b_route, W1, b1, W2, b2):
    x = inputs.reshape(T, D)
    w1c = W1.transpose(1, 0, 2).reshape(D, E * D)
    b1c = b1.reshape(1, E * D)
    w2c = W2.reshape(E * D, D)
    tril = jnp.tril(jnp.ones((CH, CH), jnp.float32))

    out = pl.pallas_call(
        _body,
        grid=(NSTEP,),
        in_specs=[
            pl.BlockSpec((CH, D), lambda i: (i, 0)),
            pl.BlockSpec((D, E), lambda i: (0, 0)),
            pl.BlockSpec((1, E), lambda i: (0, 0)),
            pl.BlockSpec((D, E * D), lambda i: (0, 0)),
            pl.BlockSpec((1, E * D), lambda i: (0, 0)),
            pl.BlockSpec((E * D, D), lambda i: (0, 0)),
            pl.BlockSpec((E, D), lambda i: (0, 0)),
            pl.BlockSpec((CH, CH), lambda i: (0, 0)),
        ],
        out_specs=pl.BlockSpec((CH, D), lambda i: (i, 0)),
        out_shape=jax.ShapeDtypeStruct((T, D), jnp.float32),
        scratch_shapes=[pltpu.VMEM((1, E), jnp.float32)],
        compiler_params=pltpu.CompilerParams(
            dimension_semantics=("arbitrary",)),
    )(x, W_route, b_route.reshape(1, E), w1c, b1c, w2c, b2, tril)
    return out.reshape(inputs.shape)
